# baseline (speedup 1.0000x reference)
import functools
import math

import jax
import jax.numpy as jnp
from jax import lax
from jax.experimental import pallas as pl
from jax.experimental.pallas import tpu as pltpu
from jax.experimental.pallas import tpu_sc as plsc

N_NODES_C = 10000
N_PAD = 10240
N_EDGES_C = 320000
HID = 256

EDGE_BLK = 1280
NODE_BLK = 2048

N_TILES = 16
C_FSPLIT = 80
C_ESPLIT = 40
G_STAGE = 10
NS = 25
ROWS_PER_TILE = N_PAD // N_TILES


def _edge_e(edge_attr, w1, b1, u, c, split):
    n_edges = edge_attr.shape[0]
    grid = n_edges // EDGE_BLK
    fdim = u.shape[1]

    def body(ea_ref, w1_ref, b1_ref, u_ref, c_ref, o_ref):
        ea = ea_ref[...]
        hid = jnp.maximum(
            jnp.dot(ea, w1_ref[...], preferred_element_type=jnp.float32)
            + b1_ref[...], 0.0)
        e = (jnp.dot(hid.astype(jnp.bfloat16), u_ref[...].astype(jnp.bfloat16),
                     preferred_element_type=jnp.float32)
             + c_ref[...])
        if split:
            o_ref[0] = e[:, :fdim // 2]
            o_ref[1] = e[:, fdim // 2:]
        else:
            o_ref[...] = e

    full = lambda shape: pl.BlockSpec(shape, lambda i: (0,) * len(shape))
    in_specs = [pl.BlockSpec((EDGE_BLK, 16), lambda i: (i, 0)),
                full(w1.shape), full(b1.shape), full(u.shape), full(c.shape)]
    if split:
        out_specs = pl.BlockSpec((2, EDGE_BLK, fdim // 2), lambda i: (0, i, 0))
        out_shape = jax.ShapeDtypeStruct((2, n_edges, fdim // 2), jnp.float32)
    else:
        out_specs = pl.BlockSpec((EDGE_BLK, fdim), lambda i: (i, 0))
        out_shape = jax.ShapeDtypeStruct((n_edges, fdim), jnp.float32)
    return pl.pallas_call(
        body, grid=(grid,), in_specs=in_specs, out_specs=out_specs,
        out_shape=out_shape)(edge_attr, w1, b1, u, c)


def _sc_agg(h_in, e_in, src3, dst3, esplit):
    f = h_in.shape[-1]
    nf16 = f // 16
    c_sz = C_ESPLIT if esplit else C_FSPLIT
    n_chunk = NS * G_STAGE
    mesh = plsc.VectorSubcoreMesh(core_axis_name="c", subcore_axis_name="s")

    @functools.partial(
        pl.kernel,
        out_type=jax.ShapeDtypeStruct((2, N_PAD, f), jnp.float32),
        mesh=mesh,
        scratch_types=[
            pltpu.VMEM((G_STAGE, 1, c_sz), jnp.int32),
            pltpu.VMEM((G_STAGE, 1, c_sz), jnp.int32),
            pltpu.VMEM((G_STAGE, 1, c_sz), jnp.int32),
            pltpu.VMEM((G_STAGE, 1, c_sz), jnp.int32),
            pltpu.VMEM((c_sz, f), jnp.float32),
            pltpu.VMEM((c_sz, f), jnp.float32),
            pltpu.VMEM((c_sz, f), jnp.float32),
            pltpu.VMEM((c_sz, f), jnp.float32),
            pltpu.VMEM_SHARED((N_PAD, f), jnp.float32),
            pltpu.SemaphoreType.DMA,
            pltpu.SemaphoreType.DMA,
            pltpu.SemaphoreType.DMA,
            pltpu.SemaphoreType.DMA,
            pltpu.SemaphoreType.DMA,
            pltpu.SemaphoreType.DMA,
        ],
    )
    def k(h_hbm, e_hbm, src_hbm, dst_hbm, out_hbm,
          ss0, ss1, ds0, ds1, h0, h1, e0, e1, aggs,
          sm0, sm1, hs0, hs1, es0, es1):
        cid = lax.axis_index("c")
        sid = lax.axis_index("s")
        sstg = (ss0, ss1)
        dstg = (ds0, ds1)
        hbufs = (h0, h1)
        ebufs = (e0, e1)
        ssems = (sm0, sm1)
        hsems = (hs0, hs1)
        esems = (es0, es1)
        if esplit:
            h_view = h_hbm
            e_view = e_hbm
            wid = cid * N_TILES + sid
        else:
            h_view = h_hbm.at[cid]
            e_view = e_hbm.at[cid]
            wid = sid
        cbase = wid * n_chunk

        zeros16 = jnp.zeros((16,), jnp.float32)

        def zrow(j, carry):
            for ff in range(nf16):
                e0[j, pl.ds(ff * 16, 16)] = zeros16
            return carry

        lax.fori_loop(0, c_sz, zrow, 0)
        row0 = sid * ROWS_PER_TILE

        def zcopy(q, carry):
            pltpu.sync_copy(e0, aggs.at[pl.ds(row0 + q * c_sz, c_sz)])
            return carry

        lax.fori_loop(0, ROWS_PER_TILE // c_sz, zcopy, 0)
        plsc.subcore_barrier()

        def issue_stage(si, sb):
            @pl.when(si < NS)
            def _():
                off = cbase + si * G_STAGE
                pltpu.async_copy(src_hbm.at[pl.ds(off, G_STAGE)],
                                 sstg[sb], ssems[sb])
                pltpu.async_copy(dst_hbm.at[pl.ds(off, G_STAGE)],
                                 dstg[sb], ssems[sb])

        def wait_stage(si, sb):
            off = cbase + si * G_STAGE
            pltpu.make_async_copy(src_hbm.at[pl.ds(off, G_STAGE)],
                                  sstg[sb], ssems[sb]).wait()
            pltpu.make_async_copy(dst_hbm.at[pl.ds(off, G_STAGE)],
                                  dstg[sb], ssems[sb]).wait()

        def issue_data(kk, g, sb, db):
            pltpu.async_copy(h_view.at[sstg[sb].at[g, 0]], hbufs[db],
                             hsems[db])
            pltpu.async_copy(e_view.at[pl.ds((cbase + kk) * c_sz, c_sz)],
                             ebufs[db], esems[db])

        def consume_data(kk, g, sb, db):
            pltpu.make_async_copy(h_view.at[sstg[sb].at[g, 0]], hbufs[db],
                                  hsems[db]).wait()
            pltpu.make_async_copy(
                e_view.at[pl.ds((cbase + kk) * c_sz, c_sz)], ebufs[db],
                esems[db]).wait()
            hb, eb = hbufs[db], ebufs[db]

            def ew(j, carry):
                for ff in range(nf16):
                    sl = pl.ds(ff * 16, 16)
                    eb[j, sl] = jnp.maximum(hb[j, sl] + eb[j, sl], 0.0)
                return carry

            lax.fori_loop(0, c_sz, ew, 0)
            pltpu.sync_copy(eb, aggs.at[dstg[sb].at[g, 0]], add=True)

        def emit_stage(si, sb):
            wait_stage(si, sb)
            issue_stage(si + 1, 1 - sb)
            k0 = si * G_STAGE
            issue_data(k0, 0, sb, 0)

            def gp(t, carry):
                g0 = 2 * t
                issue_data(k0 + g0 + 1, g0 + 1, sb, 1)
                consume_data(k0 + g0, g0, sb, 0)

                @pl.when(g0 + 2 < G_STAGE)
                def _():
                    issue_data(k0 + g0 + 2, g0 + 2, sb, 0)

                consume_data(k0 + g0 + 1, g0 + 1, sb, 1)
                return carry

            lax.fori_loop(0, G_STAGE // 2, gp, 0)

        issue_stage(0, 0)

        def pair(t, carry):
            emit_stage(2 * t, 0)
            emit_stage(2 * t + 1, 1)
            return carry

        lax.fori_loop(0, NS // 2, pair, 0)
        if NS % 2:
            emit_stage(NS - 1, 0)

        plsc.subcore_barrier()
        pltpu.sync_copy(aggs.at[pl.ds(row0, ROWS_PER_TILE)],
                        out_hbm.at[cid, pl.ds(row0, ROWS_PER_TILE)])

    return k(h_in, e_in, src3, dst3)


def _node_mlp(h_arr, agg_split, w1, b1, w2, b2, scale, shift, esplit):
    n = agg_split.shape[1]
    fin = agg_split.shape[2]
    grid = n // NODE_BLK

    def body(h_ref, a_ref, w1_ref, b1_ref, w2_ref, b2_ref, s_ref, t_ref,
             o_ref):
        if esplit:
            z = h_ref[...] + a_ref[0] + a_ref[1]
        else:
            z = jnp.concatenate([h_ref[0] + a_ref[0], h_ref[1] + a_ref[1]],
                                axis=1)
        y = jnp.maximum(
            jnp.dot(z.astype(jnp.bfloat16), w1_ref[...].astype(jnp.bfloat16),
                    preferred_element_type=jnp.float32)
            + b1_ref[...], 0.0)
        y = jnp.dot(y.astype(jnp.bfloat16), w2_ref[...].astype(jnp.bfloat16),
                    preferred_element_type=jnp.float32) + b2_ref[...]
        y = jnp.maximum(y, 0.0)
        y = y * s_ref[...] + t_ref[...]
        o_ref[0] = y[:, :HID // 2]
        o_ref[1] = y[:, HID // 2:]

    full = lambda shape: pl.BlockSpec(shape, lambda i: (0,) * len(shape))
    h_spec = (pl.BlockSpec((NODE_BLK, fin), lambda i: (i, 0)) if esplit
              else pl.BlockSpec((2, NODE_BLK, fin), lambda i: (0, i, 0)))
    in_specs = [h_spec,
                pl.BlockSpec((2, NODE_BLK, fin), lambda i: (0, i, 0)),
                full(w1.shape), full(b1.shape), full(w2.shape), full(b2.shape),
                full(scale.shape), full(shift.shape)]
    return pl.pallas_call(
        body, grid=(grid,), in_specs=in_specs,
        out_specs=pl.BlockSpec((2, NODE_BLK, HID // 2), lambda i: (0, i, 0)),
        out_shape=jax.ShapeDtypeStruct((2, n, HID // 2), jnp.float32),
    )(h_arr, agg_split, w1, b1, w2, b2, scale, shift)


def _tail(h_split, batch2d, lattice, lw1, lb1, ls, lt, lw2, lb2,
          fw1, fb1, fs, ft, fw2, fb2, ngraphs):
    n = h_split.shape[1]

    def body(h_ref, b_ref, lat_ref, lw1_ref, lb1_ref, ls_ref, lt_ref,
             lw2_ref, lb2_ref, fw1_ref, fb1_ref, fs_ref, ft_ref,
             fw2_ref, fb2_ref, o_ref):
        h = jnp.concatenate([h_ref[0], h_ref[1]], axis=1)
        b = b_ref[...]
        gids = lax.broadcasted_iota(jnp.int32, (ngraphs, n), 0)
        onehot = (gids == jnp.broadcast_to(b, (ngraphs, n))).astype(jnp.float32)
        sums = jnp.dot(onehot, h, preferred_element_type=jnp.float32)
        cnt = jnp.sum(onehot, axis=1, keepdims=True)
        pool = sums / jnp.maximum(cnt, 1.0)
        lat = lat_ref[...]
        lf = jnp.maximum(
            jnp.dot(lat, lw1_ref[...], preferred_element_type=jnp.float32)
            + lb1_ref[...], 0.0)
        lf = lf * ls_ref[...] + lt_ref[...]
        lf = jnp.dot(lf, lw2_ref[...], preferred_element_type=jnp.float32) + lb2_ref[...]
        cat = jnp.concatenate([pool, lf], axis=1)
        y = jnp.maximum(
            jnp.dot(cat, fw1_ref[...], preferred_element_type=jnp.float32)
            + fb1_ref[...], 0.0)
        y = y * fs_ref[...] + ft_ref[...]
        o_ref[...] = (jnp.dot(y, fw2_ref[...], preferred_element_type=jnp.float32)
                      + fb2_ref[...])

    args = (h_split, batch2d, lattice, lw1, lb1, ls, lt, lw2, lb2,
            fw1, fb1, fs, ft, fw2, fb2)
    return pl.pallas_call(
        body,
        out_shape=jax.ShapeDtypeStruct((ngraphs, fw2.shape[1]), jnp.float32),
    )(*args)


def kernel(x, edge_attr, lattice, params, edge_index, batch):
    p = params
    num_gnn = 4
    bn_scale = 1.0 / math.sqrt(1.0 + 1e-5)

    us, cs = [], []
    for i in range(num_gnn):
        we = p[f"g{i}_We"]
        us.append(p["ee_W2"] @ we)
        cs.append(p["ee_b2"] @ we + p[f"g{i}_be"])

    src16 = edge_index[0].reshape(-1, 1, C_FSPLIT)
    dst16 = edge_index[1].reshape(-1, 1, C_FSPLIT)
    src32 = edge_index[0].reshape(-1, 1, C_ESPLIT)
    dst32 = edge_index[1].reshape(-1, 1, C_ESPLIT)

    xp = jnp.pad(x, ((0, N_PAD - x.shape[0]), (0, 0)))
    e0 = _edge_e(edge_attr, p["ee_W1"], p["ee_b1"], us[0], cs[0], split=False)
    agg0 = _sc_agg(xp, e0, src32, dst32, esplit=True)
    h_split = _node_mlp(xp, agg0, p["g0_W1"], p["g0_b1"],
                        p["g0_W2"], p["g0_b2"],
                        p["g0_g"] * bn_scale, p["g0_bt"], esplit=True)
    for i in range(1, num_gnn):
        e_i = _edge_e(edge_attr, p["ee_W1"], p["ee_b1"], us[i], cs[i],
                      split=True)
        agg_split = _sc_agg(h_split, e_i, src16, dst16, esplit=False)
        h_split = _node_mlp(h_split, agg_split,
                            p[f"g{i}_W1"], p[f"g{i}_b1"],
                            p[f"g{i}_W2"], p[f"g{i}_b2"],
                            p[f"g{i}_g"] * bn_scale, p[f"g{i}_bt"],
                            esplit=False)

    ngraphs = lattice.shape[0]
    batch_pad = jnp.pad(batch, (0, N_PAD - batch.shape[0]),
                        constant_values=ngraphs)
    out = _tail(h_split, batch_pad.reshape(1, -1), lattice.reshape(ngraphs, 9),
                p["lat_W1"], p["lat_b1"], p["lat_g"] * bn_scale, p["lat_bt"],
                p["lat_W2"], p["lat_b2"],
                p["f_W1"], p["f_b1"], p["f_g"] * bn_scale, p["f_bt"],
                p["f_W2"], p["f_b2"], ngraphs)
    return out


# speedup vs baseline: 1.0914x; 1.0914x over previous
import functools
import math

import numpy as np

import jax
import jax.numpy as jnp
from jax import lax
from jax.experimental import pallas as pl
from jax.experimental.pallas import tpu as pltpu
from jax.experimental.pallas import tpu_sc as plsc

N_NODES_C = 10000
N_PAD = 10240
N_EDGES_C = 320000
HID = 256

EDGE_BLK = 1280
NODE_BLK = 2048

N_TILES = 16
C_EDGE = 80
NS = 25
ROWS_PER_TILE = N_PAD // N_TILES


def _pos_perm(n_workers, n_chunk):
    hblk = EDGE_BLK // 2
    rows = np.arange(n_workers * n_chunk * (C_EDGE // 2), dtype=np.int32)
    ea = (rows // hblk) * EDGE_BLK + rows % hblk
    r3 = (n_workers, n_chunk, C_EDGE // 2)
    chunk = np.concatenate([ea.reshape(r3), (ea + hblk).reshape(r3)], axis=2)
    return chunk.reshape(-1)


def _pack_i32(a, b):
    ab = lax.bitcast_convert_type(
        a.astype(jnp.bfloat16).astype(jnp.float32), jnp.int32)
    bb = lax.bitcast_convert_type(
        b.astype(jnp.bfloat16).astype(jnp.float32), jnp.int32)
    return lax.bitwise_or(lax.shift_right_logical(ab, 16),
                          lax.bitwise_and(bb, jnp.int32(-65536)))


def _edge_e(edge_attr, w1, b1, u, c, split):
    n_edges = edge_attr.shape[0]
    grid = n_edges // EDGE_BLK
    fdim = u.shape[1]
    hblk = EDGE_BLK // 2

    def body(ea_ref, w1_ref, b1_ref, u_ref, c_ref, o_ref):
        ea = ea_ref[...]
        hid = jnp.maximum(
            jnp.dot(ea, w1_ref[...], preferred_element_type=jnp.float32)
            + b1_ref[...], 0.0)
        e = (jnp.dot(hid.astype(jnp.bfloat16), u_ref[...].astype(jnp.bfloat16),
                     preferred_element_type=jnp.float32)
             + c_ref[...])
        if split:
            for cc in range(2):
                half = e[:, cc * 128:(cc + 1) * 128]
                pa = _pack_i32(half[:hblk, :64], half[:hblk, 64:])
                pb = _pack_i32(half[hblk:, :64], half[hblk:, 64:])
                o_ref[cc] = jnp.concatenate([pa, pb], axis=1)
        else:
            pa = _pack_i32(e[:hblk, :64], e[:hblk, 64:])
            pb = _pack_i32(e[hblk:, :64], e[hblk:, 64:])
            o_ref[...] = jnp.concatenate([pa, pb], axis=1)

    full = lambda shape: pl.BlockSpec(shape, lambda i: (0,) * len(shape))
    in_specs = [pl.BlockSpec((EDGE_BLK, 16), lambda i: (i, 0)),
                full(w1.shape), full(b1.shape), full(u.shape), full(c.shape)]
    if split:
        out_specs = pl.BlockSpec((2, hblk, 128), lambda i: (0, i, 0))
        out_shape = jax.ShapeDtypeStruct((2, n_edges // 2, 128), jnp.int32)
    else:
        out_specs = pl.BlockSpec((hblk, 128), lambda i: (i, 0))
        out_shape = jax.ShapeDtypeStruct((n_edges // 2, 128), jnp.int32)
    return pl.pallas_call(
        body, grid=(grid,), in_specs=in_specs, out_specs=out_specs,
        out_shape=out_shape)(edge_attr, w1, b1, u, c)


def _sc_agg(h_in, e_in, src3, dst3, esplit):
    f = 128
    nf16 = 4
    c_sz = C_EDGE
    hrow = c_sz // 2
    nw = 32 if esplit else 16
    gst = 5 if esplit else 10
    n_chunk = src3.shape[0] // nw
    mesh = plsc.VectorSubcoreMesh(core_axis_name="c", subcore_axis_name="s")

    @functools.partial(
        pl.kernel,
        out_type=jax.ShapeDtypeStruct((2, N_PAD, f), jnp.float32),
        mesh=mesh,
        scratch_types=[
            pltpu.VMEM((gst, 1, c_sz), jnp.int32),
            pltpu.VMEM((gst, 1, c_sz), jnp.int32),
            pltpu.VMEM((gst, 1, c_sz), jnp.int32),
            pltpu.VMEM((gst, 1, c_sz), jnp.int32),
            pltpu.VMEM((c_sz, f), jnp.float32),
            pltpu.VMEM((c_sz, f), jnp.float32),
            pltpu.VMEM((hrow, f), jnp.int32),
            pltpu.VMEM((hrow, f), jnp.int32),
            pltpu.VMEM_SHARED((N_PAD, f), jnp.float32),
            pltpu.SemaphoreType.DMA,
            pltpu.SemaphoreType.DMA,
            pltpu.SemaphoreType.DMA,
            pltpu.SemaphoreType.DMA,
            pltpu.SemaphoreType.DMA,
            pltpu.SemaphoreType.DMA,
        ],
    )
    def k(h_hbm, e_hbm, src_hbm, dst_hbm, out_hbm,
          ss0, ss1, ds0, ds1, h0, h1, e0, e1, aggs,
          sm0, sm1, hs0, hs1, es0, es1):
        cid = lax.axis_index("c")
        sid = lax.axis_index("s")
        sstg = (ss0, ss1)
        dstg = (ds0, ds1)
        hbufs = (h0, h1)
        ebufs = (e0, e1)
        ssems = (sm0, sm1)
        hsems = (hs0, hs1)
        esems = (es0, es1)
        if esplit:
            h_view = h_hbm
            e_view = e_hbm
            wid = cid * N_TILES + sid
        else:
            h_view = h_hbm.at[cid]
            e_view = e_hbm.at[cid]
            wid = sid
        cbase = wid * n_chunk
        rbase = cbase * hrow

        zeros16 = jnp.zeros((16,), jnp.float32)

        def zrow(j, carry):
            for ff in range(f // 16):
                h0[j, pl.ds(ff * 16, 16)] = zeros16
            return carry

        lax.fori_loop(0, c_sz, zrow, 0)
        row0 = sid * ROWS_PER_TILE

        def zcopy(q, carry):
            pltpu.sync_copy(h0, aggs.at[pl.ds(row0 + q * c_sz, c_sz)])
            return carry

        lax.fori_loop(0, ROWS_PER_TILE // c_sz, zcopy, 0)
        plsc.subcore_barrier()

        def issue_stage(si, sb):
            @pl.when(si < NS)
            def _():
                off = cbase + si * gst
                pltpu.async_copy(src_hbm.at[pl.ds(off, gst)],
                                 sstg[sb], ssems[sb])
                pltpu.async_copy(dst_hbm.at[pl.ds(off, gst)],
                                 dstg[sb], ssems[sb])

        def wait_stage(si, sb):
            off = cbase + si * gst
            pltpu.make_async_copy(src_hbm.at[pl.ds(off, gst)],
                                  sstg[sb], ssems[sb]).wait()
            pltpu.make_async_copy(dst_hbm.at[pl.ds(off, gst)],
                                  dstg[sb], ssems[sb]).wait()

        def issue_data(kk, g, sb, db):
            pltpu.async_copy(h_view.at[sstg[sb].at[g, 0]], hbufs[db],
                             hsems[db])
            pltpu.async_copy(e_view.at[pl.ds(rbase + kk * hrow, hrow)],
                             ebufs[db], esems[db])

        def consume_data(kk, g, sb, db):
            pltpu.make_async_copy(h_view.at[sstg[sb].at[g, 0]], hbufs[db],
                                  hsems[db]).wait()
            pltpu.make_async_copy(
                e_view.at[pl.ds(rbase + kk * hrow, hrow)], ebufs[db],
                esems[db]).wait()
            hb, eb = hbufs[db], ebufs[db]

            def ew(j, carry):
                for q in range(nf16):
                    sl = pl.ds(q * 16, 16)
                    sh = pl.ds(64 + q * 16, 16)
                    va = eb[j, sl]
                    ea_lo = lax.bitcast_convert_type(
                        lax.shift_left(va, 16), jnp.float32)
                    ea_hi = lax.bitcast_convert_type(
                        lax.bitwise_and(va, jnp.int32(-65536)), jnp.float32)
                    hb[j, sl] = jnp.maximum(hb[j, sl] + ea_lo, 0.0)
                    hb[j, sh] = jnp.maximum(hb[j, sh] + ea_hi, 0.0)
                    vb = eb[j, sh]
                    eb_lo = lax.bitcast_convert_type(
                        lax.shift_left(vb, 16), jnp.float32)
                    eb_hi = lax.bitcast_convert_type(
                        lax.bitwise_and(vb, jnp.int32(-65536)), jnp.float32)
                    jb = j + hrow
                    hb[jb, sl] = jnp.maximum(hb[jb, sl] + eb_lo, 0.0)
                    hb[jb, sh] = jnp.maximum(hb[jb, sh] + eb_hi, 0.0)
                return carry

            lax.fori_loop(0, hrow, ew, 0)
            pltpu.sync_copy(hb, aggs.at[dstg[sb].at[g, 0]], add=True)

        def emit_stage(si, sb):
            wait_stage(si, sb)
            issue_stage(si + 1, 1 - sb)
            k0 = si * gst
            issue_data(k0, 0, sb, 0)

            def gp(t, carry):
                g0 = 2 * t
                issue_data(k0 + g0 + 1, g0 + 1, sb, 1)
                consume_data(k0 + g0, g0, sb, 0)

                @pl.when(g0 + 2 < gst)
                def _():
                    issue_data(k0 + g0 + 2, g0 + 2, sb, 0)

                consume_data(k0 + g0 + 1, g0 + 1, sb, 1)
                return carry

            lax.fori_loop(0, gst // 2, gp, 0)
            if gst % 2:
                consume_data(k0 + gst - 1, gst - 1, sb, 0)

        issue_stage(0, 0)

        def pair(t, carry):
            emit_stage(2 * t, 0)
            emit_stage(2 * t + 1, 1)
            return carry

        lax.fori_loop(0, NS // 2, pair, 0)
        if NS % 2:
            emit_stage(NS - 1, 0)

        plsc.subcore_barrier()
        pltpu.sync_copy(aggs.at[pl.ds(row0, ROWS_PER_TILE)],
                        out_hbm.at[cid, pl.ds(row0, ROWS_PER_TILE)])

    return k(h_in, e_in, src3, dst3)


def _node_mlp(h_arr, agg_split, w1, b1, w2, b2, scale, shift, esplit):
    n = agg_split.shape[1]
    fin = agg_split.shape[2]
    grid = n // NODE_BLK

    def body(h_ref, a_ref, w1_ref, b1_ref, w2_ref, b2_ref, s_ref, t_ref,
             o_ref):
        if esplit:
            z = h_ref[...] + a_ref[0] + a_ref[1]
        else:
            z = jnp.concatenate([h_ref[0] + a_ref[0], h_ref[1] + a_ref[1]],
                                axis=1)
        y = jnp.maximum(
            jnp.dot(z.astype(jnp.bfloat16), w1_ref[...].astype(jnp.bfloat16),
                    preferred_element_type=jnp.float32)
            + b1_ref[...], 0.0)
        y = jnp.dot(y.astype(jnp.bfloat16), w2_ref[...].astype(jnp.bfloat16),
                    preferred_element_type=jnp.float32) + b2_ref[...]
        y = jnp.maximum(y, 0.0)
        y = y * s_ref[...] + t_ref[...]
        o_ref[0] = y[:, :HID // 2]
        o_ref[1] = y[:, HID // 2:]

    full = lambda shape: pl.BlockSpec(shape, lambda i: (0,) * len(shape))
    h_spec = (pl.BlockSpec((NODE_BLK, fin), lambda i: (i, 0)) if esplit
              else pl.BlockSpec((2, NODE_BLK, fin), lambda i: (0, i, 0)))
    in_specs = [h_spec,
                pl.BlockSpec((2, NODE_BLK, fin), lambda i: (0, i, 0)),
                full(w1.shape), full(b1.shape), full(w2.shape), full(b2.shape),
                full(scale.shape), full(shift.shape)]
    return pl.pallas_call(
        body, grid=(grid,), in_specs=in_specs,
        out_specs=pl.BlockSpec((2, NODE_BLK, HID // 2), lambda i: (0, i, 0)),
        out_shape=jax.ShapeDtypeStruct((2, n, HID // 2), jnp.float32),
    )(h_arr, agg_split, w1, b1, w2, b2, scale, shift)


def _tail(h_split, batch2d, lattice, lw1, lb1, ls, lt, lw2, lb2,
          fw1, fb1, fs, ft, fw2, fb2, ngraphs):
    n = h_split.shape[1]

    def body(h_ref, b_ref, lat_ref, lw1_ref, lb1_ref, ls_ref, lt_ref,
             lw2_ref, lb2_ref, fw1_ref, fb1_ref, fs_ref, ft_ref,
             fw2_ref, fb2_ref, o_ref):
        h = jnp.concatenate([h_ref[0], h_ref[1]], axis=1)
        b = b_ref[...]
        gids = lax.broadcasted_iota(jnp.int32, (ngraphs, n), 0)
        onehot = (gids == jnp.broadcast_to(b, (ngraphs, n))).astype(jnp.float32)
        sums = jnp.dot(onehot, h, preferred_element_type=jnp.float32)
        cnt = jnp.sum(onehot, axis=1, keepdims=True)
        pool = sums / jnp.maximum(cnt, 1.0)
        lat = lat_ref[...]
        lf = jnp.maximum(
            jnp.dot(lat, lw1_ref[...], preferred_element_type=jnp.float32)
            + lb1_ref[...], 0.0)
        lf = lf * ls_ref[...] + lt_ref[...]
        lf = jnp.dot(lf, lw2_ref[...], preferred_element_type=jnp.float32) + lb2_ref[...]
        cat = jnp.concatenate([pool, lf], axis=1)
        y = jnp.maximum(
            jnp.dot(cat, fw1_ref[...], preferred_element_type=jnp.float32)
            + fb1_ref[...], 0.0)
        y = y * fs_ref[...] + ft_ref[...]
        o_ref[...] = (jnp.dot(y, fw2_ref[...], preferred_element_type=jnp.float32)
                      + fb2_ref[...])

    args = (h_split, batch2d, lattice, lw1, lb1, ls, lt, lw2, lb2,
            fw1, fb1, fs, ft, fw2, fb2)
    return pl.pallas_call(
        body,
        out_shape=jax.ShapeDtypeStruct((ngraphs, fw2.shape[1]), jnp.float32),
    )(*args)


def kernel(x, edge_attr, lattice, params, edge_index, batch):
    p = params
    num_gnn = 4
    bn_scale = 1.0 / math.sqrt(1.0 + 1e-5)

    us, cs = [], []
    for i in range(num_gnn):
        we = p[f"g{i}_We"]
        us.append(p["ee_W2"] @ we)
        cs.append(p["ee_b2"] @ we + p[f"g{i}_be"])

    permf = jnp.asarray(_pos_perm(N_TILES, NS * 10))
    perme = jnp.asarray(_pos_perm(2 * N_TILES, NS * 5))
    src16 = edge_index[0][permf].reshape(-1, 1, C_EDGE)
    dst16 = edge_index[1][permf].reshape(-1, 1, C_EDGE)
    src32 = edge_index[0][perme].reshape(-1, 1, C_EDGE)
    dst32 = edge_index[1][perme].reshape(-1, 1, C_EDGE)

    xp = jnp.pad(x, ((0, N_PAD - x.shape[0]), (0, 0)))
    e0 = _edge_e(edge_attr, p["ee_W1"], p["ee_b1"], us[0], cs[0], split=False)
    agg0 = _sc_agg(xp, e0, src32, dst32, esplit=True)
    h_split = _node_mlp(xp, agg0, p["g0_W1"], p["g0_b1"],
                        p["g0_W2"], p["g0_b2"],
                        p["g0_g"] * bn_scale, p["g0_bt"], esplit=True)
    for i in range(1, num_gnn):
        e_i = _edge_e(edge_attr, p["ee_W1"], p["ee_b1"], us[i], cs[i],
                      split=True)
        agg_split = _sc_agg(h_split, e_i, src16, dst16, esplit=False)
        h_split = _node_mlp(h_split, agg_split,
                            p[f"g{i}_W1"], p[f"g{i}_b1"],
                            p[f"g{i}_W2"], p[f"g{i}_b2"],
                            p[f"g{i}_g"] * bn_scale, p[f"g{i}_bt"],
                            esplit=False)

    ngraphs = lattice.shape[0]
    batch_pad = jnp.pad(batch, (0, N_PAD - batch.shape[0]),
                        constant_values=ngraphs)
    out = _tail(h_split, batch_pad.reshape(1, -1), lattice.reshape(ngraphs, 9),
                p["lat_W1"], p["lat_b1"], p["lat_g"] * bn_scale, p["lat_bt"],
                p["lat_W2"], p["lat_b2"],
                p["f_W1"], p["f_b1"], p["f_g"] * bn_scale, p["f_bt"],
                p["f_W2"], p["f_b2"], ngraphs)
    return out


# speedup vs baseline: 1.0977x; 1.0058x over previous
import functools
import math

import numpy as np

import jax
import jax.numpy as jnp
from jax import lax
from jax.experimental import pallas as pl
from jax.experimental.pallas import tpu as pltpu
from jax.experimental.pallas import tpu_sc as plsc

N_NODES_C = 10000
N_PAD = 10240
N_EDGES_C = 320000
HID = 256

EDGE_BLK = 1280
NODE_BLK = 2048

N_TILES = 16
C_EDGE = 80
NS = 25
ROWS_PER_TILE = N_PAD // N_TILES


def _pack_i32(a, b):
    ab = lax.bitcast_convert_type(
        a.astype(jnp.bfloat16).astype(jnp.float32), jnp.int32)
    bb = lax.bitcast_convert_type(
        b.astype(jnp.bfloat16).astype(jnp.float32), jnp.int32)
    return lax.bitwise_or(lax.shift_right_logical(ab, 16),
                          lax.bitwise_and(bb, jnp.int32(-65536)))


def _edge_e(edge_attr, w1, b1, u, c, split):
    n_edges = edge_attr.shape[0]
    grid = n_edges // EDGE_BLK
    fdim = u.shape[1]
    hblk = EDGE_BLK // 2

    def body(ea_ref, w1_ref, b1_ref, u_ref, c_ref, o_ref):
        ea = ea_ref[...]
        hid = jnp.maximum(
            jnp.dot(ea, w1_ref[...], preferred_element_type=jnp.float32)
            + b1_ref[...], 0.0)
        e = (jnp.dot(hid.astype(jnp.bfloat16), u_ref[...].astype(jnp.bfloat16),
                     preferred_element_type=jnp.float32)
             + c_ref[...])
        if split:
            for cc in range(2):
                half = e[:, cc * 128:(cc + 1) * 128]
                pa = _pack_i32(half[:hblk, :64], half[:hblk, 64:])
                pb = _pack_i32(half[hblk:, :64], half[hblk:, 64:])
                o_ref[cc] = jnp.concatenate([pa, pb], axis=1)
        else:
            pa = _pack_i32(e[:hblk, :64], e[:hblk, 64:])
            pb = _pack_i32(e[hblk:, :64], e[hblk:, 64:])
            o_ref[...] = jnp.concatenate([pa, pb], axis=1)

    full = lambda shape: pl.BlockSpec(shape, lambda i: (0,) * len(shape))
    in_specs = [pl.BlockSpec((EDGE_BLK, 16), lambda i: (i, 0)),
                full(w1.shape), full(b1.shape), full(u.shape), full(c.shape)]
    if split:
        out_specs = pl.BlockSpec((2, hblk, 128), lambda i: (0, i, 0))
        out_shape = jax.ShapeDtypeStruct((2, n_edges // 2, 128), jnp.int32)
    else:
        out_specs = pl.BlockSpec((hblk, 128), lambda i: (i, 0))
        out_shape = jax.ShapeDtypeStruct((n_edges // 2, 128), jnp.int32)
    return pl.pallas_call(
        body, grid=(grid,), in_specs=in_specs, out_specs=out_specs,
        out_shape=out_shape)(edge_attr, w1, b1, u, c)


def _sc_agg(h_in, e_in, src3, dst3, esplit):
    f = 128
    nf16 = 4
    c_sz = C_EDGE
    hrow = c_sz // 2
    nw = 32 if esplit else 16
    gst = 5 if esplit else 10
    n_chunk = src3.shape[0] // nw
    mesh = plsc.VectorSubcoreMesh(core_axis_name="c", subcore_axis_name="s")

    @functools.partial(
        pl.kernel,
        out_type=jax.ShapeDtypeStruct((2, N_PAD, f), jnp.float32),
        mesh=mesh,
        scratch_types=[
            pltpu.VMEM((gst, 1, c_sz), jnp.int32),
            pltpu.VMEM((gst, 1, c_sz), jnp.int32),
            pltpu.VMEM((gst, 1, c_sz), jnp.int32),
            pltpu.VMEM((gst, 1, c_sz), jnp.int32),
            pltpu.VMEM((c_sz, f), jnp.float32),
            pltpu.VMEM((c_sz, f), jnp.float32),
            pltpu.VMEM((hrow, f), jnp.int32),
            pltpu.VMEM((hrow, f), jnp.int32),
            pltpu.VMEM_SHARED((N_PAD, f), jnp.float32),
            pltpu.SemaphoreType.DMA,
            pltpu.SemaphoreType.DMA,
            pltpu.SemaphoreType.DMA,
            pltpu.SemaphoreType.DMA,
            pltpu.SemaphoreType.DMA,
            pltpu.SemaphoreType.DMA,
        ],
    )
    def k(h_hbm, e_hbm, src_hbm, dst_hbm, out_hbm,
          ss0, ss1, ds0, ds1, h0, h1, e0, e1, aggs,
          sm0, sm1, hs0, hs1, es0, es1):
        cid = lax.axis_index("c")
        sid = lax.axis_index("s")
        sstg = (ss0, ss1)
        dstg = (ds0, ds1)
        hbufs = (h0, h1)
        ebufs = (e0, e1)
        ssems = (sm0, sm1)
        hsems = (hs0, hs1)
        esems = (es0, es1)
        if esplit:
            h_view = h_hbm
            e_view = e_hbm
            wid = cid * N_TILES + sid
        else:
            h_view = h_hbm.at[cid]
            e_view = e_hbm.at[cid]
            wid = sid
        cbase = wid * n_chunk
        if esplit:
            blk0 = cid * n_chunk
        else:
            blk0 = 0
        srow = sid * hrow

        zeros16 = jnp.zeros((16,), jnp.float32)

        def zrow(j, carry):
            for ff in range(f // 16):
                h0[j, pl.ds(ff * 16, 16)] = zeros16
            return carry

        lax.fori_loop(0, c_sz, zrow, 0)
        row0 = sid * ROWS_PER_TILE

        def zcopy(q, carry):
            pltpu.sync_copy(h0, aggs.at[pl.ds(row0 + q * c_sz, c_sz)])
            return carry

        lax.fori_loop(0, ROWS_PER_TILE // c_sz, zcopy, 0)
        plsc.subcore_barrier()

        def issue_stage(si, sb):
            @pl.when(si < NS)
            def _():
                off = cbase + si * gst
                pltpu.async_copy(src_hbm.at[pl.ds(off, gst)],
                                 sstg[sb], ssems[sb])
                pltpu.async_copy(dst_hbm.at[pl.ds(off, gst)],
                                 dstg[sb], ssems[sb])

        def wait_stage(si, sb):
            off = cbase + si * gst
            pltpu.make_async_copy(src_hbm.at[pl.ds(off, gst)],
                                  sstg[sb], ssems[sb]).wait()
            pltpu.make_async_copy(dst_hbm.at[pl.ds(off, gst)],
                                  dstg[sb], ssems[sb]).wait()

        def erows(kk):
            return pl.ds((blk0 + kk) * (EDGE_BLK // 2) + srow, hrow)

        def issue_data(kk, g, sb, db):
            pltpu.async_copy(h_view.at[sstg[sb].at[g, 0]], hbufs[db],
                             hsems[db])
            pltpu.async_copy(e_view.at[erows(kk)], ebufs[db], esems[db])

        def consume_data(kk, g, sb, db):
            pltpu.make_async_copy(h_view.at[sstg[sb].at[g, 0]], hbufs[db],
                                  hsems[db]).wait()
            pltpu.make_async_copy(e_view.at[erows(kk)], ebufs[db],
                                  esems[db]).wait()
            hb, eb = hbufs[db], ebufs[db]

            def ew(j, carry):
                for q in range(nf16):
                    sl = pl.ds(q * 16, 16)
                    sh = pl.ds(64 + q * 16, 16)
                    va = eb[j, sl]
                    ea_lo = lax.bitcast_convert_type(
                        lax.shift_left(va, 16), jnp.float32)
                    ea_hi = lax.bitcast_convert_type(
                        lax.bitwise_and(va, jnp.int32(-65536)), jnp.float32)
                    hb[j, sl] = jnp.maximum(hb[j, sl] + ea_lo, 0.0)
                    hb[j, sh] = jnp.maximum(hb[j, sh] + ea_hi, 0.0)
                    vb = eb[j, sh]
                    eb_lo = lax.bitcast_convert_type(
                        lax.shift_left(vb, 16), jnp.float32)
                    eb_hi = lax.bitcast_convert_type(
                        lax.bitwise_and(vb, jnp.int32(-65536)), jnp.float32)
                    jb = j + hrow
                    hb[jb, sl] = jnp.maximum(hb[jb, sl] + eb_lo, 0.0)
                    hb[jb, sh] = jnp.maximum(hb[jb, sh] + eb_hi, 0.0)
                return carry

            lax.fori_loop(0, hrow, ew, 0)
            pltpu.sync_copy(hb, aggs.at[dstg[sb].at[g, 0]], add=True)

        def emit_stage(si, sb):
            wait_stage(si, sb)
            issue_stage(si + 1, 1 - sb)
            k0 = si * gst
            issue_data(k0, 0, sb, 0)

            def gp(t, carry):
                g0 = 2 * t
                issue_data(k0 + g0 + 1, g0 + 1, sb, 1)
                consume_data(k0 + g0, g0, sb, 0)

                @pl.when(g0 + 2 < gst)
                def _():
                    issue_data(k0 + g0 + 2, g0 + 2, sb, 0)

                consume_data(k0 + g0 + 1, g0 + 1, sb, 1)
                return carry

            lax.fori_loop(0, gst // 2, gp, 0)
            if gst % 2:
                consume_data(k0 + gst - 1, gst - 1, sb, 0)

        issue_stage(0, 0)

        def pair(t, carry):
            emit_stage(2 * t, 0)
            emit_stage(2 * t + 1, 1)
            return carry

        lax.fori_loop(0, NS // 2, pair, 0)
        if NS % 2:
            emit_stage(NS - 1, 0)

        plsc.subcore_barrier()
        pltpu.sync_copy(aggs.at[pl.ds(row0, ROWS_PER_TILE)],
                        out_hbm.at[cid, pl.ds(row0, ROWS_PER_TILE)])

    return k(h_in, e_in, src3, dst3)


def _node_mlp(h_arr, agg_split, w1, b1, w2, b2, scale, shift, esplit):
    n = agg_split.shape[1]
    fin = agg_split.shape[2]
    grid = n // NODE_BLK

    def body(h_ref, a_ref, w1_ref, b1_ref, w2_ref, b2_ref, s_ref, t_ref,
             o_ref):
        if esplit:
            z = h_ref[...] + a_ref[0] + a_ref[1]
        else:
            z = jnp.concatenate([h_ref[0] + a_ref[0], h_ref[1] + a_ref[1]],
                                axis=1)
        y = jnp.maximum(
            jnp.dot(z.astype(jnp.bfloat16), w1_ref[...].astype(jnp.bfloat16),
                    preferred_element_type=jnp.float32)
            + b1_ref[...], 0.0)
        y = jnp.dot(y.astype(jnp.bfloat16), w2_ref[...].astype(jnp.bfloat16),
                    preferred_element_type=jnp.float32) + b2_ref[...]
        y = jnp.maximum(y, 0.0)
        y = y * s_ref[...] + t_ref[...]
        o_ref[0] = y[:, :HID // 2]
        o_ref[1] = y[:, HID // 2:]

    full = lambda shape: pl.BlockSpec(shape, lambda i: (0,) * len(shape))
    h_spec = (pl.BlockSpec((NODE_BLK, fin), lambda i: (i, 0)) if esplit
              else pl.BlockSpec((2, NODE_BLK, fin), lambda i: (0, i, 0)))
    in_specs = [h_spec,
                pl.BlockSpec((2, NODE_BLK, fin), lambda i: (0, i, 0)),
                full(w1.shape), full(b1.shape), full(w2.shape), full(b2.shape),
                full(scale.shape), full(shift.shape)]
    return pl.pallas_call(
        body, grid=(grid,), in_specs=in_specs,
        out_specs=pl.BlockSpec((2, NODE_BLK, HID // 2), lambda i: (0, i, 0)),
        out_shape=jax.ShapeDtypeStruct((2, n, HID // 2), jnp.float32),
    )(h_arr, agg_split, w1, b1, w2, b2, scale, shift)


def _tail(h_split, batch2d, lattice, lw1, lb1, ls, lt, lw2, lb2,
          fw1, fb1, fs, ft, fw2, fb2, ngraphs):
    n = h_split.shape[1]

    def body(h_ref, b_ref, lat_ref, lw1_ref, lb1_ref, ls_ref, lt_ref,
             lw2_ref, lb2_ref, fw1_ref, fb1_ref, fs_ref, ft_ref,
             fw2_ref, fb2_ref, o_ref):
        h = jnp.concatenate([h_ref[0], h_ref[1]], axis=1)
        b = b_ref[...]
        gids = lax.broadcasted_iota(jnp.int32, (ngraphs, n), 0)
        onehot = (gids == jnp.broadcast_to(b, (ngraphs, n))).astype(jnp.float32)
        sums = jnp.dot(onehot, h, preferred_element_type=jnp.float32)
        cnt = jnp.sum(onehot, axis=1, keepdims=True)
        pool = sums / jnp.maximum(cnt, 1.0)
        lat = lat_ref[...]
        lf = jnp.maximum(
            jnp.dot(lat, lw1_ref[...], preferred_element_type=jnp.float32)
            + lb1_ref[...], 0.0)
        lf = lf * ls_ref[...] + lt_ref[...]
        lf = jnp.dot(lf, lw2_ref[...], preferred_element_type=jnp.float32) + lb2_ref[...]
        cat = jnp.concatenate([pool, lf], axis=1)
        y = jnp.maximum(
            jnp.dot(cat, fw1_ref[...], preferred_element_type=jnp.float32)
            + fb1_ref[...], 0.0)
        y = y * fs_ref[...] + ft_ref[...]
        o_ref[...] = (jnp.dot(y, fw2_ref[...], preferred_element_type=jnp.float32)
                      + fb2_ref[...])

    args = (h_split, batch2d, lattice, lw1, lb1, ls, lt, lw2, lb2,
            fw1, fb1, fs, ft, fw2, fb2)
    return pl.pallas_call(
        body,
        out_shape=jax.ShapeDtypeStruct((ngraphs, fw2.shape[1]), jnp.float32),
    )(*args)


def kernel(x, edge_attr, lattice, params, edge_index, batch):
    p = params
    num_gnn = 4
    bn_scale = 1.0 / math.sqrt(1.0 + 1e-5)

    us, cs = [], []
    for i in range(num_gnn):
        we = p[f"g{i}_We"]
        us.append(p["ee_W2"] @ we)
        cs.append(p["ee_b2"] @ we + p[f"g{i}_be"])

    def _order16(v):
        v4 = v.reshape(NS * 10, 2, N_TILES, C_EDGE // 2)
        return v4.transpose(2, 0, 1, 3).reshape(-1, 1, C_EDGE)

    def _order32(v):
        v5 = v.reshape(2, NS * 5, 2, N_TILES, C_EDGE // 2)
        return v5.transpose(0, 3, 1, 2, 4).reshape(-1, 1, C_EDGE)

    src16 = _order16(edge_index[0])
    dst16 = _order16(edge_index[1])
    src32 = _order32(edge_index[0])
    dst32 = _order32(edge_index[1])

    xp = jnp.pad(x, ((0, N_PAD - x.shape[0]), (0, 0)))
    e0 = _edge_e(edge_attr, p["ee_W1"], p["ee_b1"], us[0], cs[0], split=False)
    agg0 = _sc_agg(xp, e0, src32, dst32, esplit=True)
    h_split = _node_mlp(xp, agg0, p["g0_W1"], p["g0_b1"],
                        p["g0_W2"], p["g0_b2"],
                        p["g0_g"] * bn_scale, p["g0_bt"], esplit=True)
    for i in range(1, num_gnn):
        e_i = _edge_e(edge_attr, p["ee_W1"], p["ee_b1"], us[i], cs[i],
                      split=True)
        agg_split = _sc_agg(h_split, e_i, src16, dst16, esplit=False)
        h_split = _node_mlp(h_split, agg_split,
                            p[f"g{i}_W1"], p[f"g{i}_b1"],
                            p[f"g{i}_W2"], p[f"g{i}_b2"],
                            p[f"g{i}_g"] * bn_scale, p[f"g{i}_bt"],
                            esplit=False)

    ngraphs = lattice.shape[0]
    batch_pad = jnp.pad(batch, (0, N_PAD - batch.shape[0]),
                        constant_values=ngraphs)
    out = _tail(h_split, batch_pad.reshape(1, -1), lattice.reshape(ngraphs, 9),
                p["lat_W1"], p["lat_b1"], p["lat_g"] * bn_scale, p["lat_bt"],
                p["lat_W2"], p["lat_b2"],
                p["f_W1"], p["f_b1"], p["f_g"] * bn_scale, p["f_bt"],
                p["f_W2"], p["f_b2"], ngraphs)
    return out
